# baseline (speedup 1.0000x reference)
import functools

import jax
import jax.numpy as jnp
import numpy as np
from jax import lax
from jax.experimental import pallas as pl
from jax.experimental.pallas import tpu as pltpu
from jax.experimental.pallas import tpu_sc as plsc

NUM_BEAMS = 4
VOCAB = 32768
EOS = 2
REP = np.float32(1.2)
INV_REP = np.float32(1.0) / np.float32(1.2)
K2 = 2 * NUM_BEAMS
ROWS = 128
CUR_LEN = 50
NW = 32
RPW = ROWS // NW
CHUNKS = VOCAB // 16
GROUPS = CHUNKS // 16
CAP = 4096

_LN2 = np.float32(0.6931471805599453)
_SQRT2 = np.float32(1.4142135623730951)


def _ln16(x16):
    bits = plsc.bitcast(x16, jnp.int32)
    e = (bits >> 23) - 127
    m = plsc.bitcast((bits & jnp.int32(0x7FFFFF)) | jnp.int32(127 << 23),
                     jnp.float32)
    big = m > _SQRT2
    m = jnp.where(big, m * jnp.float32(0.5), m)
    e = jnp.where(big, e + 1, e)
    z = (m - 1.0) / (m + 1.0)
    z2 = z * z
    p = jnp.float32(1.0 / 9.0)
    p = jnp.float32(1.0 / 7.0) + z2 * p
    p = jnp.float32(1.0 / 5.0) + z2 * p
    p = jnp.float32(1.0 / 3.0) + z2 * p
    p = jnp.float32(1.0) + z2 * p
    return e.astype(jnp.float32) * _LN2 + jnp.float32(2.0) * z * p


def _sc_body(
    scores_hbm, beam_hbm, tok_hbm, nsp_hbm, ntp_hbm, newtok_hbm,
    rowb0, rowb1, tokbuf, beambuf, cv, ci, hits, gmaxes, tokstage,
    sem_a, sem_b, sem_c,
):
    cid = lax.axis_index("c")
    sid = lax.axis_index("s")
    wid = sid * 2 + cid
    base = wid * RPW

    lane = jnp.arange(16, dtype=jnp.int32)
    neg = jnp.float32(-jnp.inf)
    bufs = (rowb0, rowb1)

    pltpu.sync_copy(tok_hbm.at[pl.ds(base * 64, RPW * 64)], tokbuf)
    pltpu.sync_copy(beam_hbm, beambuf)
    bvec = beambuf[pl.ds(base, 16)]

    pending = pltpu.async_copy(scores_hbm.at[base], rowb0, sem_a)

    ys = []
    flats = []
    for r in range(RPW):
        pending.wait()
        if r < RPW - 1:
            pending = pltpu.async_copy(
                scores_hbm.at[base + r + 1],
                bufs[(r + 1) % 2],
                sem_b if r % 2 == 0 else sem_a,
            )
        row = bufs[r % 2]

        tb = r * 64
        tgs = [tokbuf[pl.ds(tb + g * 16, 16)] for g in range(4)]
        tailmask = lane < (CUR_LEN - 48)
        gvs = [plsc.load_gather(row, [tg]) for tg in tgs[:3]]
        gvs.append(plsc.load_gather(row, [tgs[3]], mask=tailmask))
        pens = [jnp.where(gv < 0.0, gv * REP, gv * INV_REP) for gv in gvs]
        for tg, pen in zip(tgs[:3], pens[:3]):
            plsc.store_scatter(row, [tg], pen)
        plsc.store_scatter(row, [tgs[3]], pens[3], mask=tailmask)

        @plsc.parallel_loop(0, GROUPS, unroll=2, carry=jnp.full((16,), neg))
        def m16(g, m):
            acc = row[pl.ds(g * 256, 16)]
            for j in range(1, 16):
                acc = jnp.maximum(acc, row[pl.ds(g * 256 + j * 16, 16)])
            gmaxes[pl.ds(g * 16, 16)] = acc
            return jnp.maximum(m, acc)

        m0 = jnp.max(m16)
        srt, _ = plsc.sort_key_val(m16, lane, descending=True)
        t = jnp.max(jnp.where(lane == 7, srt, neg))

        @plsc.parallel_loop(
            0, CHUNKS, unroll=8, carry=jnp.zeros((16,), jnp.float32)
        )
        def s16(i, s):
            return s + jnp.exp(row[pl.ds(i * 16, 16)] - m0)

        ssum = jnp.sum(s16)

        def hg_body(q, goff):
            b = q * 256
            macc = plsc.load_gather(gmaxes, [b + lane * 16])
            for l in range(1, 16):
                macc = jnp.maximum(
                    macc, plsc.load_gather(gmaxes, [b + lane * 16 + l])
                )
            gm = macc >= t
            plsc.store_compressed(hits.at[pl.ds(goff, 16)], q * 16 + lane, mask=gm)
            return goff + jnp.max(plsc.all_reduce_population_count(gm))

        nhits = lax.fori_loop(0, GROUPS // 16, hg_body, jnp.int32(0))

        def hit_body(h, off):
            hg = jnp.max(plsc.load_gather(hits, [jnp.full((16,), h, jnp.int32)]))
            for j in range(16):
                cvec = row[pl.ds(hg * 256 + j * 16, 16)]
                msk = cvec >= t
                off_c = jnp.minimum(off, CAP)
                plsc.store_compressed(cv.at[pl.ds(off_c, 16)], cvec, mask=msk)
                plsc.store_compressed(
                    ci.at[pl.ds(off_c, 16)], hg * 256 + j * 16 + lane, mask=msk
                )
                off = off + jnp.max(plsc.all_reduce_population_count(msk))
            return off

        ncand = lax.fori_loop(0, nhits, hit_body, jnp.int32(0))
        ncand = jnp.minimum(ncand, CAP)

        def red_body(q, carry):
            tv, ti = carry
            b = q * 16
            cvec = jnp.where(b + lane < ncand, cv[pl.ds(b, 16)], neg)
            ivec = ci[pl.ds(b, 16)]
            cs, cis = plsc.sort_key_val(cvec, ivec, descending=True)
            rt = lax.rev(tv, (0,))
            ri = lax.rev(ti, (0,))
            take = cs > rt
            mv = jnp.where(take, cs, rt)
            mi = jnp.where(take, cis, ri)
            tv2, ti2 = plsc.sort_key_val(mv, mi, descending=True)
            return tv2, ti2

        tv, ti = lax.fori_loop(
            0,
            (ncand + 15) // 16,
            red_body,
            (jnp.full((16,), neg), jnp.zeros((16,), jnp.int32)),
        )

        logs = jnp.max(_ln16(jnp.full((16,), ssum)))
        beam_r = jnp.max(jnp.where(lane == r, bvec, neg))
        ys.append(((tv - m0) - logs) + beam_r)
        flats.append(ti + jnp.int32(r * VOCAB))

    big = jnp.int32(2**30)
    sel_s, sel_f = [], []
    for _ in range(K2):
        mv = jnp.maximum(jnp.maximum(ys[0], ys[1]), jnp.maximum(ys[2], ys[3]))
        m = jnp.max(mv)
        fmv = big
        for yv, fv in zip(ys, flats):
            fmv = jnp.minimum(fmv, jnp.where(yv == m, fv, big))
        fm = jnp.min(fmv)
        for i in range(RPW):
            ys[i] = jnp.where(flats[i] == fm, neg, ys[i])
        sel_s.append(m)
        sel_f.append(fm)

    cnt = jnp.int32(0)
    zf = jnp.float32(0.0)
    zi = jnp.int32(0)
    out_s = [zf] * NUM_BEAMS
    out_t = [zi] * NUM_BEAMS
    out_b = [zi] * NUM_BEAMS
    for k in range(K2):
        tok_k = sel_f[k] & jnp.int32(VOCAB - 1)
        beam_k = sel_f[k] >> 15
        ok = tok_k != EOS
        for slot in range(NUM_BEAMS):
            put = ok & (cnt == slot)
            out_s[slot] = jnp.where(put, sel_s[k], out_s[slot])
            out_t[slot] = jnp.where(put, tok_k, out_t[slot])
            out_b[slot] = jnp.where(put, beam_k, out_b[slot])
        cnt = cnt + ok.astype(jnp.int32)

    nsv = jnp.zeros((16,), jnp.float32)
    ntv = jnp.zeros((16,), jnp.int32)
    for slot in range(NUM_BEAMS):
        nsv = jnp.where(lane == slot, out_s[slot], nsv)
        ntv = jnp.where(lane == slot, out_t[slot], ntv)
    for slot in range(NUM_BEAMS):
        for g in range(4):
            sel = tokbuf[pl.ds(g * 16, 16)]
            for w in range(1, NUM_BEAMS):
                sel = jnp.where(
                    out_b[slot] == w, tokbuf[pl.ds(w * 64 + g * 16, 16)], sel
                )
            if g == 3:
                sel = jnp.where(lane == (CUR_LEN - 48), out_t[slot], sel)
            tokstage[pl.ds(slot * 64 + g * 16, 16)] = sel

    stage = cv
    stage[pl.ds(0, 16)] = nsv
    pltpu.sync_copy(stage.at[pl.ds(0, 16)], nsp_hbm.at[pl.ds(wid * 16, 16)])
    ci[pl.ds(0, 16)] = ntv
    pltpu.sync_copy(ci.at[pl.ds(0, 16)], ntp_hbm.at[pl.ds(wid * 16, 16)])
    cps = [
        pltpu.async_copy(
            tokstage.at[pl.ds(slot * 64, 64)],
            newtok_hbm.at[pl.ds((base + slot) * 64, 64)],
            sem_c,
        )
        for slot in range(NUM_BEAMS)
    ]
    for cp in cps:
        cp.wait()


@functools.cache
def _get_sc_kernel():
    mesh = plsc.VectorSubcoreMesh(core_axis_name="c", subcore_axis_name="s")
    return pl.kernel(
        _sc_body,
        out_type=[
            jax.ShapeDtypeStruct((NW * 16,), jnp.float32),
            jax.ShapeDtypeStruct((NW * 16,), jnp.int32),
            jax.ShapeDtypeStruct((ROWS * 64,), jnp.int32),
        ],
        mesh=mesh,
        compiler_params=pltpu.CompilerParams(needs_layout_passes=False),
        scratch_types=[
            pltpu.VMEM((VOCAB,), jnp.float32),
            pltpu.VMEM((VOCAB,), jnp.float32),
            pltpu.VMEM((RPW * 64,), jnp.int32),
            pltpu.VMEM((ROWS,), jnp.float32),
            pltpu.VMEM((CAP + 16,), jnp.float32),
            pltpu.VMEM((CAP + 16,), jnp.int32),
            pltpu.VMEM((CHUNKS + 16,), jnp.int32),
            pltpu.VMEM((CHUNKS,), jnp.float32),
            pltpu.VMEM((RPW * 64,), jnp.int32),
            pltpu.SemaphoreType.DMA,
            pltpu.SemaphoreType.DMA,
            pltpu.SemaphoreType.DMA,
        ],
    )


@jax.jit
def kernel(scores, beam_scores, token_ids):
    rows, _ = scores.shape
    cur_len = token_ids.shape[1]

    tokflat = jnp.pad(token_ids, ((0, 0), (0, 64 - cur_len))).reshape(-1)
    nsp, ntp, newtok = _get_sc_kernel()(scores, beam_scores, tokflat)

    ns = nsp.reshape(-1, 16)[:, :NUM_BEAMS]
    nt = ntp.reshape(-1, 16)[:, :NUM_BEAMS]
    return (
        ns,
        nt,
        newtok.reshape(rows, 64)[:, : cur_len + 1],
        ns.reshape(rows),
    )


# speedup vs baseline: 1.1814x; 1.1814x over previous
import functools

import jax
import jax.numpy as jnp
import numpy as np
from jax import lax
from jax.experimental import pallas as pl
from jax.experimental.pallas import tpu as pltpu
from jax.experimental.pallas import tpu_sc as plsc

NUM_BEAMS = 4
VOCAB = 32768
EOS = 2
REP = np.float32(1.2)
INV_REP = np.float32(1.0) / np.float32(1.2)
K2 = 2 * NUM_BEAMS
ROWS = 128
CUR_LEN = 50
NW = 32
RPW = ROWS // NW
CHUNKS = VOCAB // 16
GVECS = 8
GROUPS = CHUNKS // GVECS
CAP = 4096

_LN2 = np.float32(0.6931471805599453)
_SQRT2 = np.float32(1.4142135623730951)


def _ln16(x16):
    bits = plsc.bitcast(x16, jnp.int32)
    e = (bits >> 23) - 127
    m = plsc.bitcast((bits & jnp.int32(0x7FFFFF)) | jnp.int32(127 << 23),
                     jnp.float32)
    big = m > _SQRT2
    m = jnp.where(big, m * jnp.float32(0.5), m)
    e = jnp.where(big, e + 1, e)
    z = (m - 1.0) / (m + 1.0)
    z2 = z * z
    p = jnp.float32(1.0 / 9.0)
    p = jnp.float32(1.0 / 7.0) + z2 * p
    p = jnp.float32(1.0 / 5.0) + z2 * p
    p = jnp.float32(1.0 / 3.0) + z2 * p
    p = jnp.float32(1.0) + z2 * p
    return e.astype(jnp.float32) * _LN2 + jnp.float32(2.0) * z * p


def _sc_body(
    scores_hbm, beam_hbm, tok_hbm, nsp_hbm, ntp_hbm, newtok_hbm,
    rowb0, rowb1, tokbuf, beambuf, cv, ci, hits, gmaxes, gsums, tokstage,
    sem_a, sem_b, sem_c,
):
    cid = lax.axis_index("c")
    sid = lax.axis_index("s")
    wid = sid * 2 + cid
    base = wid * RPW

    lane = jnp.arange(16, dtype=jnp.int32)
    neg = jnp.float32(-jnp.inf)
    bufs = (rowb0, rowb1)

    pltpu.sync_copy(tok_hbm.at[pl.ds(base * 64, RPW * 64)], tokbuf)
    pltpu.sync_copy(beam_hbm, beambuf)
    bvec = beambuf[pl.ds(base, 16)]

    pending = pltpu.async_copy(scores_hbm.at[base], rowb0, sem_a)

    ys = []
    flats = []
    for r in range(RPW):
        pending.wait()
        if r < RPW - 1:
            pending = pltpu.async_copy(
                scores_hbm.at[base + r + 1],
                bufs[(r + 1) % 2],
                sem_b if r % 2 == 0 else sem_a,
            )
        row = bufs[r % 2]

        tb = r * 64
        tgs = [tokbuf[pl.ds(tb + g * 16, 16)] for g in range(4)]
        tailmask = lane < (CUR_LEN - 48)
        gvs = [plsc.load_gather(row, [tg]) for tg in tgs[:3]]
        gvs.append(plsc.load_gather(row, [tgs[3]], mask=tailmask))
        pens = [jnp.where(gv < 0.0, gv * REP, gv * INV_REP) for gv in gvs]
        for tg, pen in zip(tgs[:3], pens[:3]):
            plsc.store_scatter(row, [tg], pen)
        plsc.store_scatter(row, [tgs[3]], pens[3], mask=tailmask)

        @plsc.parallel_loop(0, GROUPS, unroll=2, carry=jnp.full((16,), neg))
        def m16(g, m):
            vs = [row[pl.ds(g * 128 + j * 16, 16)] for j in range(GVECS)]
            acc = vs[0]
            for j in range(1, GVECS):
                acc = jnp.maximum(acc, vs[j])
            gmaxes[pl.ds(g * 16, 16)] = acc
            s = jnp.exp(vs[0] - acc)
            for j in range(1, GVECS):
                s = s + jnp.exp(vs[j] - acc)
            gsums[pl.ds(g * 16, 16)] = s
            return jnp.maximum(m, acc)

        m0 = jnp.max(m16)
        srt, _ = plsc.sort_key_val(m16, lane, descending=True)
        t = jnp.max(jnp.where(lane == 7, srt, neg))

        @plsc.parallel_loop(
            0, GROUPS, unroll=4, carry=jnp.zeros((16,), jnp.float32)
        )
        def s16(g, s):
            return s + jnp.exp(gmaxes[pl.ds(g * 16, 16)] - m0) * gsums[
                pl.ds(g * 16, 16)
            ]

        ssum = jnp.sum(s16)

        def hg_body(q, goff):
            b = q * 256
            macc = plsc.load_gather(gmaxes, [b + lane * 16])
            for l in range(1, 16):
                macc = jnp.maximum(
                    macc, plsc.load_gather(gmaxes, [b + lane * 16 + l])
                )
            gm = macc >= t
            plsc.store_compressed(hits.at[pl.ds(goff, 16)], q * 16 + lane, mask=gm)
            return goff + jnp.max(plsc.all_reduce_population_count(gm))

        nhits = lax.fori_loop(0, GROUPS // 16, hg_body, jnp.int32(0))

        def hit_body(h, off):
            hg = jnp.max(plsc.load_gather(hits, [jnp.full((16,), h, jnp.int32)]))
            for j in range(GVECS):
                cvec = row[pl.ds(hg * 128 + j * 16, 16)]
                msk = cvec >= t
                off_c = jnp.minimum(off, CAP)
                plsc.store_compressed(cv.at[pl.ds(off_c, 16)], cvec, mask=msk)
                plsc.store_compressed(
                    ci.at[pl.ds(off_c, 16)], hg * 128 + j * 16 + lane, mask=msk
                )
                off = off + jnp.max(plsc.all_reduce_population_count(msk))
            return off

        ncand = lax.fori_loop(0, nhits, hit_body, jnp.int32(0))
        ncand = jnp.minimum(ncand, CAP)

        def red_body(q, carry):
            tv, ti = carry
            b = q * 16
            cvec = jnp.where(b + lane < ncand, cv[pl.ds(b, 16)], neg)
            ivec = ci[pl.ds(b, 16)]
            cs, cis = plsc.sort_key_val(cvec, ivec, descending=True)
            rt = lax.rev(tv, (0,))
            ri = lax.rev(ti, (0,))
            take = cs > rt
            mv = jnp.where(take, cs, rt)
            mi = jnp.where(take, cis, ri)
            tv2, ti2 = plsc.sort_key_val(mv, mi, descending=True)
            return tv2, ti2

        tv, ti = lax.fori_loop(
            0,
            (ncand + 15) // 16,
            red_body,
            (jnp.full((16,), neg), jnp.zeros((16,), jnp.int32)),
        )

        logs = jnp.max(_ln16(jnp.full((16,), ssum)))
        beam_r = jnp.max(jnp.where(lane == r, bvec, neg))
        ys.append(((tv - m0) - logs) + beam_r)
        flats.append(ti + jnp.int32(r * VOCAB))

    big = jnp.int32(2**30)
    sel_s, sel_f = [], []
    for _ in range(K2):
        mv = jnp.maximum(jnp.maximum(ys[0], ys[1]), jnp.maximum(ys[2], ys[3]))
        m = jnp.max(mv)
        fmv = big
        for yv, fv in zip(ys, flats):
            fmv = jnp.minimum(fmv, jnp.where(yv == m, fv, big))
        fm = jnp.min(fmv)
        for i in range(RPW):
            ys[i] = jnp.where(flats[i] == fm, neg, ys[i])
        sel_s.append(m)
        sel_f.append(fm)

    cnt = jnp.int32(0)
    zf = jnp.float32(0.0)
    zi = jnp.int32(0)
    out_s = [zf] * NUM_BEAMS
    out_t = [zi] * NUM_BEAMS
    out_b = [zi] * NUM_BEAMS
    for k in range(K2):
        tok_k = sel_f[k] & jnp.int32(VOCAB - 1)
        beam_k = sel_f[k] >> 15
        ok = tok_k != EOS
        for slot in range(NUM_BEAMS):
            put = ok & (cnt == slot)
            out_s[slot] = jnp.where(put, sel_s[k], out_s[slot])
            out_t[slot] = jnp.where(put, tok_k, out_t[slot])
            out_b[slot] = jnp.where(put, beam_k, out_b[slot])
        cnt = cnt + ok.astype(jnp.int32)

    nsv = jnp.zeros((16,), jnp.float32)
    ntv = jnp.zeros((16,), jnp.int32)
    for slot in range(NUM_BEAMS):
        nsv = jnp.where(lane == slot, out_s[slot], nsv)
        ntv = jnp.where(lane == slot, out_t[slot], ntv)
    for slot in range(NUM_BEAMS):
        for g in range(4):
            sel = tokbuf[pl.ds(g * 16, 16)]
            for w in range(1, NUM_BEAMS):
                sel = jnp.where(
                    out_b[slot] == w, tokbuf[pl.ds(w * 64 + g * 16, 16)], sel
                )
            if g == 3:
                sel = jnp.where(lane == (CUR_LEN - 48), out_t[slot], sel)
            tokstage[pl.ds(slot * 64 + g * 16, 16)] = sel

    stage = cv
    stage[pl.ds(0, 16)] = nsv
    pltpu.sync_copy(stage.at[pl.ds(0, 16)], nsp_hbm.at[pl.ds(wid * 16, 16)])
    ci[pl.ds(0, 16)] = ntv
    pltpu.sync_copy(ci.at[pl.ds(0, 16)], ntp_hbm.at[pl.ds(wid * 16, 16)])
    cps = [
        pltpu.async_copy(
            tokstage.at[pl.ds(slot * 64, 64)],
            newtok_hbm.at[pl.ds((base + slot) * 64, 64)],
            sem_c,
        )
        for slot in range(NUM_BEAMS)
    ]
    for cp in cps:
        cp.wait()


@functools.cache
def _get_sc_kernel():
    mesh = plsc.VectorSubcoreMesh(core_axis_name="c", subcore_axis_name="s")
    return pl.kernel(
        _sc_body,
        out_type=[
            jax.ShapeDtypeStruct((NW * 16,), jnp.float32),
            jax.ShapeDtypeStruct((NW * 16,), jnp.int32),
            jax.ShapeDtypeStruct((ROWS * 64,), jnp.int32),
        ],
        mesh=mesh,
        compiler_params=pltpu.CompilerParams(needs_layout_passes=False),
        scratch_types=[
            pltpu.VMEM((VOCAB,), jnp.float32),
            pltpu.VMEM((VOCAB,), jnp.float32),
            pltpu.VMEM((RPW * 64,), jnp.int32),
            pltpu.VMEM((ROWS,), jnp.float32),
            pltpu.VMEM((CAP + 16,), jnp.float32),
            pltpu.VMEM((CAP + 16,), jnp.int32),
            pltpu.VMEM((CHUNKS + 16,), jnp.int32),
            pltpu.VMEM((GROUPS * 16,), jnp.float32),
            pltpu.VMEM((GROUPS * 16,), jnp.float32),
            pltpu.VMEM((RPW * 64,), jnp.int32),
            pltpu.SemaphoreType.DMA,
            pltpu.SemaphoreType.DMA,
            pltpu.SemaphoreType.DMA,
        ],
    )


@jax.jit
def kernel(scores, beam_scores, token_ids):
    rows, _ = scores.shape
    cur_len = token_ids.shape[1]

    tokflat = jnp.pad(token_ids, ((0, 0), (0, 64 - cur_len))).reshape(-1)
    nsp, ntp, newtok = _get_sc_kernel()(scores, beam_scores, tokflat)

    ns = nsp.reshape(-1, 16)[:, :NUM_BEAMS]
    nt = ntp.reshape(-1, 16)[:, :NUM_BEAMS]
    return (
        ns,
        nt,
        newtok.reshape(rows, 64)[:, : cur_len + 1],
        ns.reshape(rows),
    )
